# Initial kernel scaffold; baseline (speedup 1.0000x reference)
import jax
import jax.numpy as jnp
from jax.experimental import pallas as pl


def kernel(adj_index, adj_values, features, W, b):
    raise NotImplementedError("write your pallas kernel here")


# speedup vs baseline: 2.7199x; 2.7199x over previous
import functools

import jax
import jax.numpy as jnp
from jax import lax
from jax.experimental import pallas as pl
from jax.experimental.pallas import tpu as pltpu
from jax.experimental.pallas import tpu_sc as plsc

N = 10000
E = 320000
CH = 128

NC = 2
NS = 16
NW = NC * NS

LANES = 16
CHUNK = 128
EPW = 10240
STEPS = EPW // CHUNK
E_PAD = EPW * NW
NPAD = 10240
ROWS_PER_SUB = NPAD // NS


def _mm_body(x_ref, w_ref, b_ref, o_ref):
    acc = jnp.dot(x_ref[...], w_ref[...], preferred_element_type=jnp.float32)
    o_ref[...] = jnp.maximum(acc + b_ref[...], 0.0)


def _tc_project(x, w, b):
    grid = (NPAD // 2048,)
    return pl.pallas_call(
        _mm_body,
        grid=grid,
        in_specs=[
            pl.BlockSpec((2048, CH), lambda i: (i, 0)),
            pl.BlockSpec((CH, CH), lambda i: (0, 0)),
            pl.BlockSpec((1, CH), lambda i: (0, 0)),
        ],
        out_specs=pl.BlockSpec((2048, CH), lambda i: (i, 0)),
        out_shape=jax.ShapeDtypeStruct((NPAD, CH), jnp.float32),
    )(x, w, b)


def _merge_body(p_ref, o_ref):
    o_ref[...] = p_ref[0] + p_ref[1]


def _tc_merge(partials):
    grid = (NPAD // 2048,)
    return pl.pallas_call(
        _merge_body,
        grid=grid,
        in_specs=[pl.BlockSpec((2, 2048, CH), lambda i: (0, i, 0))],
        out_specs=pl.BlockSpec((2048, CH), lambda i: (i, 0)),
        out_shape=jax.ShapeDtypeStruct((NPAD, CH), jnp.float32),
    )(partials)


def _spmm_body(src_hbm, dst_hbm, val_hbm, base_hbm, out_hbm,
               src_v, dst_v, val_v, rows_v, acc_sh, sem):
    cid = lax.axis_index("c")
    sid = lax.axis_index("s")
    wid = cid * NS + sid

    pltpu.sync_copy(src_hbm.at[pl.ds(wid * STEPS, STEPS)], src_v)
    pltpu.sync_copy(dst_hbm.at[pl.ds(wid * STEPS, STEPS)], dst_v)
    pltpu.sync_copy(val_hbm.at[pl.ds(wid * STEPS, STEPS)], val_v)

    zeros16 = jnp.zeros((LANES,), jnp.float32)

    def _zrow(r, _):
        for j in range(CH // LANES):
            rows_v[r, pl.ds(j * LANES, LANES)] = zeros16
        return 0

    lax.fori_loop(0, CHUNK, _zrow, 0)
    for blk in range(ROWS_PER_SUB // CHUNK):
        pltpu.sync_copy(
            rows_v, acc_sh.at[pl.ds(sid * ROWS_PER_SUB + blk * CHUNK, CHUNK)])
    plsc.subcore_barrier()

    def _step(k, _):
        idx_row = dst_v.at[k]
        pltpu.async_copy(base_hbm.at[idx_row], rows_v, sem).wait()

        def _grp(g, _):
            vvec = val_v[k, pl.ds(g * LANES, LANES)]

            def _lane(l, _):
                bcast = vvec.at[jnp.full((LANES,), l, jnp.int32)].get(
                    mode="promise_in_bounds")
                e = g * LANES + l
                for j in range(CH // LANES):
                    sl = pl.ds(j * LANES, LANES)
                    rows_v[e, sl] = rows_v[e, sl] * bcast
                return 0

            lax.fori_loop(0, LANES, _lane, 0)
            return 0

        lax.fori_loop(0, CHUNK // LANES, _grp, 0)
        pltpu.sync_copy(rows_v, acc_sh.at[src_v.at[k]], add=True)
        return 0

    lax.fori_loop(0, STEPS, _step, 0)
    plsc.subcore_barrier()

    pltpu.sync_copy(
        acc_sh.at[pl.ds(sid * ROWS_PER_SUB, ROWS_PER_SUB)],
        out_hbm.at[cid, pl.ds(sid * ROWS_PER_SUB, ROWS_PER_SUB)])


_sc_spmm = functools.partial(
    pl.kernel,
    out_type=jax.ShapeDtypeStruct((NC, NPAD, CH), jnp.float32),
    mesh=plsc.VectorSubcoreMesh(core_axis_name="c", subcore_axis_name="s"),
    scratch_types=[
        pltpu.VMEM((STEPS, CHUNK), jnp.int32),
        pltpu.VMEM((STEPS, CHUNK), jnp.int32),
        pltpu.VMEM((STEPS, CHUNK), jnp.float32),
        pltpu.VMEM((CHUNK, CH), jnp.float32),
        pltpu.VMEM_SHARED((NPAD, CH), jnp.float32),
        pltpu.SemaphoreType.DMA,
    ],
)(_spmm_body)


@jax.jit
def kernel(adj_index, adj_values, features, W, b):
    src = adj_index[0]
    dst = adj_index[1]
    pad = E_PAD - E
    src_p = jnp.concatenate([src, jnp.zeros((pad,), jnp.int32)]
                            ).reshape(NW * STEPS, CHUNK)
    dst_p = jnp.concatenate([dst, jnp.zeros((pad,), jnp.int32)]
                            ).reshape(NW * STEPS, CHUNK)
    val_p = jnp.concatenate([adj_values, jnp.zeros((pad,), jnp.float32)]
                            ).reshape(NW * STEPS, CHUNK)
    feat_p = jnp.pad(features, ((0, NPAD - N), (0, 0)))

    base = _tc_project(feat_p, W, b)
    for _ in range(2):
        partials = _sc_spmm(src_p, dst_p, val_p, base)
        base = _tc_merge(partials)
    return base[:N]


# speedup vs baseline: 3.1505x; 1.1583x over previous
import functools

import jax
import jax.numpy as jnp
from jax import lax
from jax.experimental import pallas as pl
from jax.experimental.pallas import tpu as pltpu
from jax.experimental.pallas import tpu_sc as plsc

N = 10000
E = 320000
CH = 128

NC = 2
NS = 16
NW = NC * NS

LANES = 16
CHUNK = 128
EPW = 10240
STEPS = EPW // CHUNK
GROUP = 16
E_PAD = EPW * NW
NPAD = 10240
ROWS_PER_SUB = NPAD // NS


def _mm_body(x_ref, w_ref, b_ref, o_ref):
    acc = jnp.dot(x_ref[...], w_ref[...], preferred_element_type=jnp.float32)
    o_ref[...] = jnp.maximum(acc + b_ref[...], 0.0)


def _tc_project(x, w, b):
    grid = (NPAD // 2048,)
    return pl.pallas_call(
        _mm_body,
        grid=grid,
        in_specs=[
            pl.BlockSpec((2048, CH), lambda i: (i, 0)),
            pl.BlockSpec((CH, CH), lambda i: (0, 0)),
            pl.BlockSpec((1, CH), lambda i: (0, 0)),
        ],
        out_specs=pl.BlockSpec((2048, CH), lambda i: (i, 0)),
        out_shape=jax.ShapeDtypeStruct((NPAD, CH), jnp.float32),
    )(x, w, b)


def _merge_body(p_ref, o_ref):
    o_ref[...] = p_ref[0] + p_ref[1]


def _tc_merge(partials):
    grid = (NPAD // 2048,)
    return pl.pallas_call(
        _merge_body,
        grid=grid,
        in_specs=[pl.BlockSpec((2, 2048, CH), lambda i: (0, i, 0))],
        out_specs=pl.BlockSpec((2048, CH), lambda i: (i, 0)),
        out_shape=jax.ShapeDtypeStruct((NPAD, CH), jnp.float32),
    )(partials)


def _spmm_body(src_hbm, dst_hbm, val_hbm, base_hbm, out_hbm,
               src_v, dst_v, val_v, rows0_v, rows1_v, acc_sh, sem0, sem1):
    cid = lax.axis_index("c")
    sid = lax.axis_index("s")
    wid = cid * NS + sid

    zeros16 = jnp.zeros((LANES,), jnp.float32)

    def _zrow(r, _):
        for j in range(CH // LANES):
            rows0_v[r, pl.ds(j * LANES, LANES)] = zeros16
        return 0

    lax.fori_loop(0, CHUNK, _zrow, 0)
    for blk in range(ROWS_PER_SUB // CHUNK):
        pltpu.sync_copy(
            rows0_v, acc_sh.at[pl.ds(sid * ROWS_PER_SUB + blk * CHUNK, CHUNK)])
    plsc.subcore_barrier()

    def _scale(rows_ref, k):
        def _grp(g, _):
            vvec = val_v[k, pl.ds(g * LANES, LANES)]
            for l in range(LANES):
                bcast = vvec.at[jnp.full((LANES,), l, jnp.int32)].get(
                    mode="promise_in_bounds")
                e = g * LANES + l
                for j in range(CH // LANES):
                    sl = pl.ds(j * LANES, LANES)
                    rows_ref[e, sl] = rows_ref[e, sl] * bcast
            return 0

        lax.fori_loop(0, CHUNK // LANES, _grp, 0)

    def _gather(k, rows_ref, sem):
        return pltpu.async_copy(base_hbm.at[dst_v.at[k]], rows_ref, sem)

    def _wait(rows_ref, sem):
        pltpu.make_async_copy(base_hbm.at[dst_v.at[0]], rows_ref, sem).wait()

    def _group(gi, _):
        goff = wid * STEPS + gi * GROUP
        pltpu.sync_copy(src_hbm.at[pl.ds(goff, GROUP)], src_v)
        pltpu.sync_copy(dst_hbm.at[pl.ds(goff, GROUP)], dst_v)
        pltpu.sync_copy(val_hbm.at[pl.ds(goff, GROUP)], val_v)
        _gather(0, rows0_v, sem0)

        def _step2(i, _):
            ka = 2 * i
            kb = 2 * i + 1
            _wait(rows0_v, sem0)
            _gather(kb, rows1_v, sem1)
            _scale(rows0_v, ka)
            pltpu.sync_copy(rows0_v, acc_sh.at[src_v.at[ka]], add=True)
            _wait(rows1_v, sem1)

            @pl.when(kb + 1 < GROUP)
            def _():
                _gather(kb + 1, rows0_v, sem0)

            _scale(rows1_v, kb)
            pltpu.sync_copy(rows1_v, acc_sh.at[src_v.at[kb]], add=True)
            return 0

        lax.fori_loop(0, GROUP // 2, _step2, 0)
        return 0

    lax.fori_loop(0, STEPS // GROUP, _group, 0)
    plsc.subcore_barrier()

    pltpu.sync_copy(
        acc_sh.at[pl.ds(sid * ROWS_PER_SUB, ROWS_PER_SUB)],
        out_hbm.at[cid, pl.ds(sid * ROWS_PER_SUB, ROWS_PER_SUB)])


_sc_spmm = functools.partial(
    pl.kernel,
    out_type=jax.ShapeDtypeStruct((NC, NPAD, CH), jnp.float32),
    mesh=plsc.VectorSubcoreMesh(core_axis_name="c", subcore_axis_name="s"),
    scratch_types=[
        pltpu.VMEM((GROUP, CHUNK), jnp.int32),
        pltpu.VMEM((GROUP, CHUNK), jnp.int32),
        pltpu.VMEM((GROUP, CHUNK), jnp.float32),
        pltpu.VMEM((CHUNK, CH), jnp.float32),
        pltpu.VMEM((CHUNK, CH), jnp.float32),
        pltpu.VMEM_SHARED((NPAD, CH), jnp.float32),
        pltpu.SemaphoreType.DMA,
        pltpu.SemaphoreType.DMA,
    ],
)(_spmm_body)


@jax.jit
def kernel(adj_index, adj_values, features, W, b):
    src = adj_index[0]
    dst = adj_index[1]
    pad = E_PAD - E
    src_p = jnp.concatenate([src, jnp.zeros((pad,), jnp.int32)]
                            ).reshape(NW * STEPS, CHUNK)
    dst_p = jnp.concatenate([dst, jnp.zeros((pad,), jnp.int32)]
                            ).reshape(NW * STEPS, CHUNK)
    val_p = jnp.concatenate([adj_values, jnp.zeros((pad,), jnp.float32)]
                            ).reshape(NW * STEPS, CHUNK)
    feat_p = jnp.pad(features, ((0, NPAD - N), (0, 0)))

    base = _tc_project(feat_p, W, b)
    for _ in range(2):
        partials = _sc_spmm(src_p, dst_p, val_p, base)
        base = _tc_merge(partials)
    return base[:N]


# speedup vs baseline: 3.6271x; 1.1513x over previous
import functools

import jax
import jax.numpy as jnp
from jax import lax
from jax.experimental import pallas as pl
from jax.experimental.pallas import tpu as pltpu
from jax.experimental.pallas import tpu_sc as plsc

N = 10000
E = 320000
CH = 128
HALF = CH // 2

NC = 2
NS = 16
NW = NC * NS

LANES = 16
CHUNK = 128
EPT = 20480
STEPS = EPT // CHUNK
GROUP = 16
E_PAD = EPT * NS
NPAD = 10240
ROWS_PER_SUB = NPAD // NS


def _mm_body(x_ref, w_ref, b_ref, o_ref):
    acc = jnp.dot(x_ref[...], w_ref[...], preferred_element_type=jnp.float32)
    acc = jnp.maximum(acc + b_ref[...], 0.0)
    o_ref[0] = acc[:, :HALF]
    o_ref[1] = acc[:, HALF:]


def _tc_project(x, w, b):
    grid = (NPAD // 2048,)
    return pl.pallas_call(
        _mm_body,
        grid=grid,
        in_specs=[
            pl.BlockSpec((2048, CH), lambda i: (i, 0)),
            pl.BlockSpec((CH, CH), lambda i: (0, 0)),
            pl.BlockSpec((1, CH), lambda i: (0, 0)),
        ],
        out_specs=pl.BlockSpec((2, 2048, HALF), lambda i: (0, i, 0)),
        out_shape=jax.ShapeDtypeStruct((2, NPAD, HALF), jnp.float32),
    )(x, w, b)


def _spmm_body(src_hbm, dst_hbm, val_hbm, base_hbm, out_hbm,
               src_v, dst_v, val_v, rows0_v, rows1_v, base_sh, acc_sh,
               sem0, sem1):
    cid = lax.axis_index("c")
    sid = lax.axis_index("s")

    stripe = pl.ds(sid * ROWS_PER_SUB, ROWS_PER_SUB)
    pltpu.sync_copy(base_hbm.at[cid, stripe], base_sh.at[stripe])

    zeros16 = jnp.zeros((LANES,), jnp.float32)

    def _zrow(r, _):
        for j in range(HALF // LANES):
            rows0_v[r, pl.ds(j * LANES, LANES)] = zeros16
        return 0

    lax.fori_loop(0, CHUNK, _zrow, 0)
    for blk in range(ROWS_PER_SUB // CHUNK):
        pltpu.sync_copy(
            rows0_v, acc_sh.at[pl.ds(sid * ROWS_PER_SUB + blk * CHUNK, CHUNK)])
    plsc.subcore_barrier()

    def _scale(rows_ref, k):
        def _grp(g, _):
            vvec = val_v[k, pl.ds(g * LANES, LANES)]
            for l in range(LANES):
                bcast = vvec.at[jnp.full((LANES,), l, jnp.int32)].get(
                    mode="promise_in_bounds")
                e = g * LANES + l
                for j in range(HALF // LANES):
                    sl = pl.ds(j * LANES, LANES)
                    rows_ref[e, sl] = rows_ref[e, sl] * bcast
            return 0

        lax.fori_loop(0, CHUNK // LANES, _grp, 0)

    def _gather(k, rows_ref, sem):
        return pltpu.async_copy(base_sh.at[dst_v.at[k]], rows_ref, sem)

    def _wait(rows_ref, sem):
        pltpu.make_async_copy(base_sh.at[dst_v.at[0]], rows_ref, sem).wait()

    def _group(gi, _):
        goff = sid * STEPS + gi * GROUP
        pltpu.sync_copy(src_hbm.at[pl.ds(goff, GROUP)], src_v)
        pltpu.sync_copy(dst_hbm.at[pl.ds(goff, GROUP)], dst_v)
        pltpu.sync_copy(val_hbm.at[pl.ds(goff, GROUP)], val_v)
        _gather(0, rows0_v, sem0)

        def _step2(i, _):
            ka = 2 * i
            kb = 2 * i + 1
            _wait(rows0_v, sem0)
            _gather(kb, rows1_v, sem1)
            _scale(rows0_v, ka)
            pltpu.sync_copy(rows0_v, acc_sh.at[src_v.at[ka]], add=True)
            _wait(rows1_v, sem1)

            @pl.when(kb + 1 < GROUP)
            def _():
                _gather(kb + 1, rows0_v, sem0)

            _scale(rows1_v, kb)
            pltpu.sync_copy(rows1_v, acc_sh.at[src_v.at[kb]], add=True)
            return 0

        lax.fori_loop(0, GROUP // 2, _step2, 0)
        return 0

    lax.fori_loop(0, STEPS // GROUP, _group, 0)
    plsc.subcore_barrier()

    pltpu.sync_copy(acc_sh.at[stripe], out_hbm.at[cid, stripe])


_sc_spmm = functools.partial(
    pl.kernel,
    out_type=jax.ShapeDtypeStruct((NC, NPAD, HALF), jnp.float32),
    mesh=plsc.VectorSubcoreMesh(core_axis_name="c", subcore_axis_name="s"),
    compiler_params=pltpu.CompilerParams(use_tc_tiling_on_sc=False),
    scratch_types=[
        pltpu.VMEM((GROUP, CHUNK), jnp.int32),
        pltpu.VMEM((GROUP, CHUNK), jnp.int32),
        pltpu.VMEM((GROUP, CHUNK), jnp.float32),
        pltpu.VMEM((CHUNK, HALF), jnp.float32),
        pltpu.VMEM((CHUNK, HALF), jnp.float32),
        pltpu.VMEM_SHARED((NPAD, HALF), jnp.float32),
        pltpu.VMEM_SHARED((NPAD, HALF), jnp.float32),
        pltpu.SemaphoreType.DMA,
        pltpu.SemaphoreType.DMA,
    ],
)(_spmm_body)


@jax.jit
def kernel(adj_index, adj_values, features, W, b):
    src = adj_index[0]
    dst = adj_index[1]
    pad = E_PAD - E
    src_p = jnp.concatenate([src, jnp.zeros((pad,), jnp.int32)]
                            ).reshape(NS * STEPS, CHUNK)
    dst_p = jnp.concatenate([dst, jnp.zeros((pad,), jnp.int32)]
                            ).reshape(NS * STEPS, CHUNK)
    val_p = jnp.concatenate([adj_values, jnp.zeros((pad,), jnp.float32)]
                            ).reshape(NS * STEPS, CHUNK)
    feat_p = jnp.pad(features, ((0, NPAD - N), (0, 0)))

    halves = _tc_project(feat_p, W, b)
    for _ in range(2):
        halves = _sc_spmm(src_p, dst_p, val_p, halves)
    return jnp.concatenate([halves[0], halves[1]], axis=1)[:N]


# speedup vs baseline: 3.7951x; 1.0463x over previous
import functools

import jax
import jax.numpy as jnp
from jax import lax
from jax.experimental import pallas as pl
from jax.experimental.pallas import tpu as pltpu
from jax.experimental.pallas import tpu_sc as plsc

N = 10000
E = 320000
CH = 128
HALF = CH // 2

NC = 2
NS = 16
NW = NC * NS

LANES = 16
CHUNK = 128
EPT = 20480
STEPS = EPT // CHUNK
GROUP = 16
E_PAD = EPT * NS
NPAD = 10240
ROWS_PER_SUB = NPAD // NS


def _mm_body(x_ref, w_ref, b_ref, o_ref):
    acc = jnp.dot(x_ref[...], w_ref[...], preferred_element_type=jnp.float32)
    acc = jnp.maximum(acc + b_ref[...], 0.0)
    o_ref[0] = acc[:, :HALF]
    o_ref[1] = acc[:, HALF:]


def _tc_project(x, w, b):
    grid = (NPAD // 2048,)
    return pl.pallas_call(
        _mm_body,
        grid=grid,
        in_specs=[
            pl.BlockSpec((2048, CH), lambda i: (i, 0)),
            pl.BlockSpec((CH, CH), lambda i: (0, 0)),
            pl.BlockSpec((1, CH), lambda i: (0, 0)),
        ],
        out_specs=pl.BlockSpec((2, 2048, HALF), lambda i: (0, i, 0)),
        out_shape=jax.ShapeDtypeStruct((2, NPAD, HALF), jnp.float32),
    )(x, w, b)


def _spmm_body(src_hbm, dst_hbm, val_hbm, base_hbm, out_hbm,
               src_v, dst_v, val_v, rows0_v, rows1_v, base_sh, acc_sh,
               sem0, sem1, ssem0, ssem1):
    cid = lax.axis_index("c")
    sid = lax.axis_index("s")

    stripe = pl.ds(sid * ROWS_PER_SUB, ROWS_PER_SUB)
    pltpu.sync_copy(base_hbm.at[cid, stripe], base_sh.at[stripe])

    zeros16 = jnp.zeros((LANES,), jnp.float32)

    def _zrow(r, _):
        for j in range(HALF // LANES):
            rows0_v[r, pl.ds(j * LANES, LANES)] = zeros16
        return 0

    lax.fori_loop(0, CHUNK, _zrow, 0)
    for blk in range(ROWS_PER_SUB // CHUNK):
        pltpu.sync_copy(
            rows0_v, acc_sh.at[pl.ds(sid * ROWS_PER_SUB + blk * CHUNK, CHUNK)])
    plsc.subcore_barrier()

    def _scale(rows_ref, k):
        def _grp(g, _):
            vvec = val_v[k, pl.ds(g * LANES, LANES)]
            for l in range(LANES):
                bcast = vvec.at[jnp.full((LANES,), l, jnp.int32)].get(
                    mode="promise_in_bounds")
                e = g * LANES + l
                for j in range(HALF // LANES):
                    sl = pl.ds(j * LANES, LANES)
                    rows_ref[e, sl] = rows_ref[e, sl] * bcast
            return 0

        lax.fori_loop(0, CHUNK // LANES, _grp, 0)

    def _gather(k, rows_ref, sem):
        return pltpu.async_copy(base_sh.at[dst_v.at[k]], rows_ref, sem)

    def _gwait(rows_ref, sem):
        pltpu.make_async_copy(base_sh.at[dst_v.at[0]], rows_ref, sem).wait()

    def _scatter(k, rows_ref, sem):
        return pltpu.async_copy(rows_ref, acc_sh.at[src_v.at[k]], sem,
                                add=True)

    def _swait(rows_ref, sem):
        pltpu.make_async_copy(rows_ref, acc_sh.at[src_v.at[0]], sem).wait()

    def _group(gi, _):
        goff = sid * STEPS + gi * GROUP
        pltpu.sync_copy(src_hbm.at[pl.ds(goff, GROUP)], src_v)
        pltpu.sync_copy(dst_hbm.at[pl.ds(goff, GROUP)], dst_v)
        pltpu.sync_copy(val_hbm.at[pl.ds(goff, GROUP)], val_v)
        _gather(0, rows0_v, sem0)
        _gather(1, rows1_v, sem1)

        def _step2(i, _):
            ka = 2 * i
            kb = 2 * i + 1
            _gwait(rows0_v, sem0)
            _scale(rows0_v, ka)
            _scatter(ka, rows0_v, ssem0)
            _gwait(rows1_v, sem1)
            _scale(rows1_v, kb)
            _scatter(kb, rows1_v, ssem1)
            _swait(rows0_v, ssem0)

            @pl.when(ka + 2 < GROUP)
            def _():
                _gather(ka + 2, rows0_v, sem0)

            _swait(rows1_v, ssem1)

            @pl.when(kb + 2 < GROUP)
            def _():
                _gather(kb + 2, rows1_v, sem1)

            return 0

        lax.fori_loop(0, GROUP // 2, _step2, 0)
        return 0

    lax.fori_loop(0, STEPS // GROUP, _group, 0)
    plsc.subcore_barrier()

    pltpu.sync_copy(acc_sh.at[stripe], out_hbm.at[cid, stripe])


_sc_spmm = functools.partial(
    pl.kernel,
    out_type=jax.ShapeDtypeStruct((NC, NPAD, HALF), jnp.float32),
    mesh=plsc.VectorSubcoreMesh(core_axis_name="c", subcore_axis_name="s"),
    compiler_params=pltpu.CompilerParams(use_tc_tiling_on_sc=False),
    scratch_types=[
        pltpu.VMEM((GROUP, CHUNK), jnp.int32),
        pltpu.VMEM((GROUP, CHUNK), jnp.int32),
        pltpu.VMEM((GROUP, CHUNK), jnp.float32),
        pltpu.VMEM((CHUNK, HALF), jnp.float32),
        pltpu.VMEM((CHUNK, HALF), jnp.float32),
        pltpu.VMEM_SHARED((NPAD, HALF), jnp.float32),
        pltpu.VMEM_SHARED((NPAD, HALF), jnp.float32),
        pltpu.SemaphoreType.DMA,
        pltpu.SemaphoreType.DMA,
        pltpu.SemaphoreType.DMA,
        pltpu.SemaphoreType.DMA,
    ],
)(_spmm_body)


@jax.jit
def kernel(adj_index, adj_values, features, W, b):
    src = adj_index[0]
    dst = adj_index[1]
    pad = E_PAD - E
    src_p = jnp.concatenate([src, jnp.zeros((pad,), jnp.int32)]
                            ).reshape(NS * STEPS, CHUNK)
    dst_p = jnp.concatenate([dst, jnp.zeros((pad,), jnp.int32)]
                            ).reshape(NS * STEPS, CHUNK)
    val_p = jnp.concatenate([adj_values, jnp.zeros((pad,), jnp.float32)]
                            ).reshape(NS * STEPS, CHUNK)
    feat_p = jnp.pad(features, ((0, NPAD - N), (0, 0)))

    halves = _tc_project(feat_p, W, b)
    for _ in range(2):
        halves = _sc_spmm(src_p, dst_p, val_p, halves)
    return jnp.concatenate([halves[0], halves[1]], axis=1)[:N]


# speedup vs baseline: 4.2584x; 1.1221x over previous
import functools

import jax
import jax.numpy as jnp
from jax import lax
from jax.experimental import pallas as pl
from jax.experimental.pallas import tpu as pltpu
from jax.experimental.pallas import tpu_sc as plsc

N = 10000
E = 320000
CH = 128
HALF = CH // 2

NC = 2
NS = 16
NW = NC * NS

LANES = 16
CHUNK = 128
EPT = 20480
STEPS = EPT // CHUNK
GROUP = 32
E_PAD = EPT * NS
NPAD = 10240
ROWS_PER_SUB = NPAD // NS


def _mm_body(x_ref, w_ref, b_ref, o_ref):
    acc = jnp.dot(x_ref[...], w_ref[...], preferred_element_type=jnp.float32)
    acc = jnp.maximum(acc + b_ref[...], 0.0)
    o_ref[0] = acc[:, :HALF]
    o_ref[1] = acc[:, HALF:]


def _tc_project(x, w, b):
    grid = (NPAD // 2048,)
    return pl.pallas_call(
        _mm_body,
        grid=grid,
        in_specs=[
            pl.BlockSpec((2048, CH), lambda i: (i, 0)),
            pl.BlockSpec((CH, CH), lambda i: (0, 0)),
            pl.BlockSpec((1, CH), lambda i: (0, 0)),
        ],
        out_specs=pl.BlockSpec((2, 2048, HALF), lambda i: (0, i, 0)),
        out_shape=jax.ShapeDtypeStruct((2, NPAD, HALF), jnp.float32),
    )(x, w, b)


def _spmm_body(src_hbm, dst_hbm, val_hbm, base_hbm, out_hbm,
               src_v, dst_v, val_v, rows0_v, rows1_v, rows2_v, rows3_v,
               base_sh, acc_sh,
               sem0, sem1, sem2, sem3, ssem0, ssem1, ssem2, ssem3):
    cid = lax.axis_index("c")
    sid = lax.axis_index("s")

    stripe = pl.ds(sid * ROWS_PER_SUB, ROWS_PER_SUB)
    pltpu.sync_copy(base_hbm.at[cid, stripe], base_sh.at[stripe])

    zeros16 = jnp.zeros((LANES,), jnp.float32)

    def _zrow(r, _):
        for j in range(HALF // LANES):
            rows0_v[r, pl.ds(j * LANES, LANES)] = zeros16
        return 0

    lax.fori_loop(0, CHUNK, _zrow, 0)
    for blk in range(ROWS_PER_SUB // CHUNK):
        pltpu.sync_copy(
            rows0_v, acc_sh.at[pl.ds(sid * ROWS_PER_SUB + blk * CHUNK, CHUNK)])
    plsc.subcore_barrier()

    def _scale(rows_ref, k):
        def _grp(g, _):
            vvec = val_v[k, pl.ds(g * LANES, LANES)]
            for l in range(LANES):
                bcast = vvec.at[jnp.full((LANES,), l, jnp.int32)].get(
                    mode="promise_in_bounds")
                e = g * LANES + l
                for j in range(HALF // LANES):
                    sl = pl.ds(j * LANES, LANES)
                    rows_ref[e, sl] = rows_ref[e, sl] * bcast
            return 0

        lax.fori_loop(0, CHUNK // LANES, _grp, 0)

    bufs = (rows0_v, rows1_v, rows2_v, rows3_v)
    gsems = (sem0, sem1, sem2, sem3)
    ssems = (ssem0, ssem1, ssem2, ssem3)

    def _gather(k, rows_ref, sem):
        return pltpu.async_copy(base_sh.at[dst_v.at[k]], rows_ref, sem)

    def _gwait(rows_ref, sem):
        pltpu.make_async_copy(base_sh.at[dst_v.at[0]], rows_ref, sem).wait()

    def _scatter(k, rows_ref, sem):
        return pltpu.async_copy(rows_ref, acc_sh.at[src_v.at[k]], sem,
                                add=True)

    def _swait(rows_ref, sem):
        pltpu.make_async_copy(rows_ref, acc_sh.at[src_v.at[0]], sem).wait()

    def _group(gi, _):
        goff = sid * STEPS + gi * GROUP
        pltpu.sync_copy(src_hbm.at[pl.ds(goff, GROUP)], src_v)
        pltpu.sync_copy(dst_hbm.at[pl.ds(goff, GROUP)], dst_v)
        pltpu.sync_copy(val_hbm.at[pl.ds(goff, GROUP)], val_v)
        _gather(0, rows0_v, sem0)
        _gather(1, rows1_v, sem1)

        def _quad(q, _):
            for j in range(4):
                k = 4 * q + j
                j2 = (j + 2) % 4
                _gwait(bufs[j], gsems[j])
                _scale(bufs[j], k)
                _scatter(k, bufs[j], ssems[j])

                @pl.when(jnp.logical_and(k + 2 < GROUP, k >= 2))
                def _():
                    _swait(bufs[j2], ssems[j2])

                @pl.when(k + 2 < GROUP)
                def _():
                    _gather(k + 2, bufs[j2], gsems[j2])

            return 0

        lax.fori_loop(0, GROUP // 4, _quad, 0)
        for j in range(4):
            _swait(bufs[j], ssems[j])
        return 0

    lax.fori_loop(0, STEPS // GROUP, _group, 0)
    plsc.subcore_barrier()

    pltpu.sync_copy(acc_sh.at[stripe], out_hbm.at[cid, stripe])


_sc_spmm = functools.partial(
    pl.kernel,
    out_type=jax.ShapeDtypeStruct((NC, NPAD, HALF), jnp.float32),
    mesh=plsc.VectorSubcoreMesh(core_axis_name="c", subcore_axis_name="s"),
    compiler_params=pltpu.CompilerParams(use_tc_tiling_on_sc=False),
    scratch_types=[
        pltpu.VMEM((GROUP, CHUNK), jnp.int32),
        pltpu.VMEM((GROUP, CHUNK), jnp.int32),
        pltpu.VMEM((GROUP, CHUNK), jnp.float32),
        pltpu.VMEM((CHUNK, HALF), jnp.float32),
        pltpu.VMEM((CHUNK, HALF), jnp.float32),
        pltpu.VMEM((CHUNK, HALF), jnp.float32),
        pltpu.VMEM((CHUNK, HALF), jnp.float32),
        pltpu.VMEM_SHARED((NPAD, HALF), jnp.float32),
        pltpu.VMEM_SHARED((NPAD, HALF), jnp.float32),
        pltpu.SemaphoreType.DMA,
        pltpu.SemaphoreType.DMA,
        pltpu.SemaphoreType.DMA,
        pltpu.SemaphoreType.DMA,
        pltpu.SemaphoreType.DMA,
        pltpu.SemaphoreType.DMA,
        pltpu.SemaphoreType.DMA,
        pltpu.SemaphoreType.DMA,
    ],
)(_spmm_body)


@jax.jit
def kernel(adj_index, adj_values, features, W, b):
    src = adj_index[0]
    dst = adj_index[1]
    pad = E_PAD - E
    src_p = jnp.concatenate([src, jnp.zeros((pad,), jnp.int32)]
                            ).reshape(NS * STEPS, CHUNK)
    dst_p = jnp.concatenate([dst, jnp.zeros((pad,), jnp.int32)]
                            ).reshape(NS * STEPS, CHUNK)
    val_p = jnp.concatenate([adj_values, jnp.zeros((pad,), jnp.float32)]
                            ).reshape(NS * STEPS, CHUNK)
    feat_p = jnp.pad(features, ((0, NPAD - N), (0, 0)))

    halves = _tc_project(feat_p, W, b)
    for _ in range(2):
        halves = _sc_spmm(src_p, dst_p, val_p, halves)
    return jnp.concatenate([halves[0], halves[1]], axis=1)[:N]
